# baseline (speedup 1.0000x reference)
import jax
import jax.numpy as jnp
from jax import lax
from jax.experimental import pallas as pl
from jax.experimental.pallas import tpu as pltpu
from jax.experimental.pallas import tpu_sc as plsc

_L = 16
_N = 12


def _compact_body(t2_hbm, st_hbm, out_hbm, t2_v, st_v, out_v):
    pltpu.sync_copy(t2_hbm, t2_v.at[pl.ds(0, _N)])
    pltpu.sync_copy(st_hbm, st_v.at[pl.ds(0, _N)])
    t = t2_v[...]
    s = st_v[...]
    lane = lax.iota(jnp.int32, _L)
    m = (t < 1.0) & (lane < _N)
    fill = plsc.load_gather(st_v, [jnp.zeros((_L,), jnp.int32)])
    out_v[...] = fill
    plsc.store_compressed(out_v.at[...], s, mask=m)
    pltpu.sync_copy(out_v.at[pl.ds(0, _N)], out_hbm)


def kernel(t2, stored_tensor):
    mesh = plsc.VectorSubcoreMesh(
        core_axis_name="c", subcore_axis_name="s", num_cores=1, num_subcores=1
    )
    run = pl.kernel(
        _compact_body,
        mesh=mesh,
        out_type=jax.ShapeDtypeStruct((_N,), jnp.float32),
        scratch_types=[
            pltpu.VMEM((_L,), jnp.float32),
            pltpu.VMEM((_L,), jnp.float32),
            pltpu.VMEM((_L,), jnp.float32),
        ],
        compiler_params=pltpu.CompilerParams(needs_layout_passes=False),
    )
    return run(t2.reshape(-1), stored_tensor.reshape(-1))


# speedup vs baseline: 1.0584x; 1.0584x over previous
import jax
import jax.numpy as jnp
from jax import lax
from jax.experimental import pallas as pl
from jax.experimental.pallas import tpu as pltpu
from jax.experimental.pallas import tpu_sc as plsc

_N = 12


def _compact_body(t2_hbm, st_hbm, out_hbm, t2_s, st_s, out_s):
    pltpu.sync_copy(t2_hbm, t2_s)
    pltpu.sync_copy(st_hbm, st_s)

    def step(i, cnt):
        ok = t2_s[i] < 1.0

        @pl.when(ok)
        def _():
            out_s[cnt] = st_s[i]

        return cnt + jnp.where(ok, 1, 0)

    cnt = lax.fori_loop(0, _N, step, 0)

    def pad_step(j, c):
        @pl.when(j >= c)
        def _():
            out_s[j] = st_s[0]

        return c

    lax.fori_loop(0, _N, pad_step, cnt)
    pltpu.sync_copy(out_s, out_hbm)


def kernel(t2, stored_tensor):
    mesh = plsc.ScalarSubcoreMesh(axis_name="c", num_cores=1)
    run = pl.kernel(
        _compact_body,
        mesh=mesh,
        out_type=jax.ShapeDtypeStruct((_N,), jnp.float32),
        scratch_types=[
            pltpu.SMEM((_N,), jnp.float32),
            pltpu.SMEM((_N,), jnp.float32),
            pltpu.SMEM((_N,), jnp.float32),
        ],
        compiler_params=pltpu.CompilerParams(needs_layout_passes=False),
    )
    return run(t2.reshape(-1), stored_tensor.reshape(-1))


# speedup vs baseline: 1.0640x; 1.0053x over previous
import jax
import jax.numpy as jnp
from jax.experimental import pallas as pl
from jax.experimental.pallas import tpu as pltpu
from jax.experimental.pallas import tpu_sc as plsc

_SHAPE = (2, 2, 3)
_N = 12


def _compact_body(t2_hbm, st_hbm, out_hbm, t2_s, st_s, out_s):
    pltpu.sync_copy(t2_hbm, t2_s)
    pltpu.sync_copy(st_hbm, st_s)

    st0 = st_s[0, 0, 0]
    for j in range(_N):
        out_s[j] = st0

    cnt = jnp.int32(0)
    for a in range(_SHAPE[0]):
        for b in range(_SHAPE[1]):
            for c in range(_SHAPE[2]):
                ok = t2_s[a, b, c] < 1.0

                @pl.when(ok)
                def _(a=a, b=b, c=c, cnt=cnt):
                    out_s[cnt] = st_s[a, b, c]

                cnt = cnt + jnp.where(ok, 1, 0)

    pltpu.sync_copy(out_s, out_hbm)


def kernel(t2, stored_tensor):
    mesh = plsc.ScalarSubcoreMesh(axis_name="c", num_cores=1)
    run = pl.kernel(
        _compact_body,
        mesh=mesh,
        out_type=jax.ShapeDtypeStruct((_N,), jnp.float32),
        scratch_types=[
            pltpu.SMEM(_SHAPE, jnp.float32),
            pltpu.SMEM(_SHAPE, jnp.float32),
            pltpu.SMEM((_N,), jnp.float32),
        ],
        compiler_params=pltpu.CompilerParams(needs_layout_passes=False),
    )
    return run(t2, stored_tensor)
